# baseline (speedup 1.0000x reference)
import functools
import jax
import jax.numpy as jnp
from jax import lax
from jax.experimental import pallas as pl
from jax.experimental.pallas import tpu as pltpu
from jax.experimental.pallas import tpu_sc as plsc

M = 2048
BS = 8
BM = M // BS
NW = 32
RPW = M // NW
CH = 8
LANES = 16

_SEG_STARTS = (0, 384, 640, 896, 1152, 1408, 1664, 1920)
_SEG_RUNS = (24, 16, 16, 16, 16, 16, 16, 8)


def _hist_body(img_hbm, part_hbm, imgbuf, hist, redbuf, sem0, sem1):
    ci = lax.axis_index("c")
    si = lax.axis_index("s")
    w = si * 2 + ci
    row0 = w * RPW
    lane = lax.iota(jnp.int32, LANES)
    laneoff = lane * 2048
    ones = jnp.ones((LANES,), jnp.float32)
    zeros = jnp.zeros((LANES,), jnp.float32)
    sems = (sem0, sem1)
    n_ch = RPW // CH

    handles = [None, None]
    handles[0] = pltpu.async_copy(
        img_hbm.at[pl.ds(row0, CH)], imgbuf.at[0], sems[0])

    def zero_body(t):
        hist[pl.ds(t * LANES, LANES)] = zeros

    plsc.parallel_loop(0, 32768 // LANES)(zero_body)

    for ch in range(n_ch):
        par = ch & 1
        if ch + 1 < n_ch:
            handles[1 - par] = pltpu.async_copy(
                img_hbm.at[pl.ds(row0 + (ch + 1) * CH, CH)],
                imgbuf.at[1 - par], sems[1 - par])
        handles[par].wait()

        def rs_body(t, par=par):
            row = t >> 3
            col0 = (t & 7) << 8
            svec = laneoff + col0
            for k in range(16):
                v = imgbuf[par, row, pl.ds(col0 + k * LANES, LANES)]
                plsc.addupdate_scatter(hist, [v + svec], ones)

        plsc.parallel_loop(0, CH * BS, unroll=4)(rs_body)

    for seg in range(BS):
        def red_body(c16, _):
            base = seg * 256 + c16 * LANES
            acc = hist[pl.ds(base, LANES)]
            for k in range(1, LANES):
                acc = acc + hist[pl.ds(k * 2048 + base, LANES)]
            redbuf[seg, pl.ds(c16 * LANES, LANES)] = acc
            return 0

        lax.fori_loop(0, 256 // LANES, red_body, 0)

    pltpu.sync_copy(redbuf, part_hbm.at[w % 4, pl.ds((w // 4) * BS, BS)])


_hist_kernel = pl.kernel(
    _hist_body,
    out_type=jax.ShapeDtypeStruct((4, 64, 256), jnp.float32),
    mesh=plsc.VectorSubcoreMesh(core_axis_name="c", subcore_axis_name="s"),
    scratch_types=[
        pltpu.VMEM((2, CH, 2048), jnp.int32),
        pltpu.VMEM((32768,), jnp.float32),
        pltpu.VMEM((BS, 256), jnp.float32),
        pltpu.SemaphoreType.DMA,
        pltpu.SemaphoreType.DMA,
    ],
    compiler_params=pltpu.CompilerParams(needs_layout_passes=False),
)


def _maps_body(part_ref, maps_ref):
    p = part_ref[...]
    h = p[0] + p[1] + p[2] + p[3]
    all_sum = jnp.sum(h, axis=1, keepdims=True)
    thr = 10.0 * all_sum / 256.0
    total_extra = jnp.sum(jnp.maximum(h - thr, 0.0), axis=1, keepdims=True)
    mean_extra = total_extra / 256.0
    cliph = jnp.floor(jnp.minimum(h, thr) + mean_extra)
    ri = lax.broadcasted_iota(jnp.int32, (256, 256), 0)
    cj = lax.broadcasted_iota(jnp.int32, (256, 256), 1)
    tri = (ri <= cj).astype(jnp.float32)
    cdf = jnp.dot(cliph, tri, preferred_element_type=jnp.float32)
    maps = jnp.mod(jnp.floor(cdf * (255.0 / 65536.0)), 256.0)
    shifted = jnp.concatenate([maps[8:], maps[56:64]], axis=0)
    lo = lax.bitcast_convert_type(maps.astype(jnp.bfloat16), jnp.uint16)
    hi = lax.bitcast_convert_type(shifted.astype(jnp.bfloat16), jnp.uint16)
    maps_ref[...] = lo.astype(jnp.int32) | (hi.astype(jnp.int32) << 16)


_maps_call = pl.pallas_call(
    _maps_body,
    out_shape=jax.ShapeDtypeStruct((64, 256), jnp.int32),
)


def _interp_body(img_hbm, maps_hbm, out_hbm, mapsv, imgbuf, outbuf,
                 semm, si0, si1, so0, so1):
    ci = lax.axis_index("c")
    si = lax.axis_index("s")
    w = si * 2 + ci
    row0 = w * RPW
    n_ch = RPW // CH
    sin = (si0, si1)
    sout = (so0, so1)
    hmaps = pltpu.async_copy(maps_hbm, mapsv, semm)
    lane = lax.iota(jnp.int32, LANES)
    lanef = lane.astype(jnp.float32) * (1.0 / 256.0)
    y1_seg = [lanef + (_SEG_STARTS[s] / 256.0 - (s + 0.5)) for s in range(7)]

    hin = [None, None]
    hout = [None, None]
    hin[0] = pltpu.async_copy(
        img_hbm.at[pl.ds(row0, CH)], imgbuf.at[0], sin[0])
    hmaps.wait()

    for ch in range(n_ch):
        par = ch & 1
        i0 = row0 + ch * CH
        if ch + 1 < n_ch:
            hin[1 - par] = pltpu.async_copy(
                img_hbm.at[pl.ds(row0 + (ch + 1) * CH, CH)],
                imgbuf.at[1 - par], sin[1 - par])
        hin[par].wait()
        if ch >= 2:
            hout[par].wait()

        def row_body(r8, _, par=par, i0=i0):
            i = i0 + r8
            r = jnp.maximum(i - 128, 0) >> 8
            rv = lax.broadcast(r * 2048, (LANES,))
            x1s = i - (r * 256 + 128)
            x1v = lax.broadcast(x1s, (LANES,)).astype(jnp.float32) * (1.0 / 256.0)
            redge = lax.broadcast(i >= 1920, (LANES,))
            x1v = jnp.where(redge, jnp.zeros((LANES,), jnp.float32), x1v)
            ex1 = 1.0 - x1v

            for seg in range(BS):
                start = _SEG_STARTS[seg]
                rvc0 = rv + seg * 256
                if seg < 7:
                    rvc1 = rv + (seg + 1) * 256

                    def run_body(t, y1v, start=start, rvc0=rvc0,
                                 rvc1=rvc1, par=par):
                        jb = start + t * LANES
                        v = imgbuf[par, r8, pl.ds(jb, LANES)]
                        g0 = plsc.load_gather(mapsv, [v + rvc0])
                        g1 = plsc.load_gather(mapsv, [v + rvc1])
                        lu, lb = plsc.unpack(
                            plsc.bitcast(g0, jnp.bfloat16),
                            format=plsc.PackFormat.INTERLEAVED)
                        ru, rb = plsc.unpack(
                            plsc.bitcast(g1, jnp.bfloat16),
                            format=plsc.PackFormat.INTERLEAVED)
                        t0 = ex1 * lu + x1v * lb
                        t1 = ex1 * ru + x1v * rb
                        res = (1.0 - y1v) * t0 + y1v * t1
                        q = res.astype(jnp.int32) & 255
                        outbuf[par, r8, pl.ds(jb, LANES)] = q.astype(jnp.float32)
                        return y1v + (LANES / 256.0)

                    plsc.parallel_loop(0, _SEG_RUNS[seg], unroll=8,
                                       carry=y1_seg[seg])(run_body)
                else:
                    def run_body7(t, start=start, rvc0=rvc0, par=par):
                        jb = start + t * LANES
                        v = imgbuf[par, r8, pl.ds(jb, LANES)]
                        g0 = plsc.load_gather(mapsv, [v + rvc0])
                        lu, lb = plsc.unpack(
                            plsc.bitcast(g0, jnp.bfloat16),
                            format=plsc.PackFormat.INTERLEAVED)
                        res = ex1 * lu + x1v * lb
                        q = res.astype(jnp.int32) & 255
                        outbuf[par, r8, pl.ds(jb, LANES)] = q.astype(jnp.float32)

                    plsc.parallel_loop(0, _SEG_RUNS[seg], unroll=8)(run_body7)
            return 0

        lax.fori_loop(0, CH, row_body, 0)
        hout[par] = pltpu.async_copy(
            outbuf.at[par], out_hbm.at[pl.ds(i0, CH)], sout[par])

    hout[(n_ch - 2) & 1].wait()
    hout[(n_ch - 1) & 1].wait()


_interp_kernel = pl.kernel(
    _interp_body,
    out_type=jax.ShapeDtypeStruct((M, M), jnp.float32),
    mesh=plsc.VectorSubcoreMesh(core_axis_name="c", subcore_axis_name="s"),
    scratch_types=[
        pltpu.VMEM((16384,), jnp.int32),
        pltpu.VMEM((2, CH, 2048), jnp.int32),
        pltpu.VMEM((2, CH, 2048), jnp.float32),
        pltpu.SemaphoreType.DMA,
        pltpu.SemaphoreType.DMA,
        pltpu.SemaphoreType.DMA,
        pltpu.SemaphoreType.DMA,
        pltpu.SemaphoreType.DMA,
    ],
    compiler_params=pltpu.CompilerParams(needs_layout_passes=False),
)


@jax.jit
def _clahe(img):
    partials = _hist_kernel(img)
    maps = _maps_call(partials)
    return _interp_kernel(img, maps.reshape(16384))


def kernel(img_arr, level, blocks):
    return _clahe(img_arr.astype(jnp.int32))


# speedup vs baseline: 1.0112x; 1.0112x over previous
import functools
import jax
import jax.numpy as jnp
from jax import lax
from jax.experimental import pallas as pl
from jax.experimental.pallas import tpu as pltpu
from jax.experimental.pallas import tpu_sc as plsc

M = 2048
BS = 8
BM = M // BS
NW = 32
RPW = M // NW
CH = 8
LANES = 16

_SEG_STARTS = (0, 384, 640, 896, 1152, 1408, 1664, 1920)
_SEG_RUNS = (24, 16, 16, 16, 16, 16, 16, 8)


def _hist_body(img_hbm, part_hbm, imgbuf, hist, redbuf, sem0, sem1):
    ci = lax.axis_index("c")
    si = lax.axis_index("s")
    w = si * 2 + ci
    row0 = w * RPW
    lane = lax.iota(jnp.int32, LANES)
    laneoff = lane * 2048
    ones = jnp.ones((LANES,), jnp.float32)
    zeros = jnp.zeros((LANES,), jnp.float32)
    sems = (sem0, sem1)
    n_ch = RPW // CH

    handles = [None, None]
    handles[0] = pltpu.async_copy(
        img_hbm.at[pl.ds(row0, CH)], imgbuf.at[0], sems[0])

    def zero_body(t):
        hist[pl.ds(t * LANES, LANES)] = zeros

    plsc.parallel_loop(0, 32768 // LANES)(zero_body)

    for ch in range(n_ch):
        par = ch & 1
        if ch + 1 < n_ch:
            handles[1 - par] = pltpu.async_copy(
                img_hbm.at[pl.ds(row0 + (ch + 1) * CH, CH)],
                imgbuf.at[1 - par], sems[1 - par])
        handles[par].wait()

        def rs_body(t, par=par):
            row = t >> 3
            col0 = (t & 7) << 8
            svec = laneoff + col0
            for k in range(16):
                v = imgbuf[par, row, pl.ds(col0 + k * LANES, LANES)]
                plsc.addupdate_scatter(hist, [v + svec], ones)

        plsc.parallel_loop(0, CH * BS, unroll=4)(rs_body)

    for seg in range(BS):
        def red_body(c16, _):
            base = seg * 256 + c16 * LANES
            acc = hist[pl.ds(base, LANES)]
            for k in range(1, LANES):
                acc = acc + hist[pl.ds(k * 2048 + base, LANES)]
            redbuf[seg, pl.ds(c16 * LANES, LANES)] = acc
            return 0

        lax.fori_loop(0, 256 // LANES, red_body, 0)

    pltpu.sync_copy(redbuf, part_hbm.at[w % 4, pl.ds((w // 4) * BS, BS)])


_hist_kernel = pl.kernel(
    _hist_body,
    out_type=jax.ShapeDtypeStruct((4, 64, 256), jnp.float32),
    mesh=plsc.VectorSubcoreMesh(core_axis_name="c", subcore_axis_name="s"),
    scratch_types=[
        pltpu.VMEM((2, CH, 2048), jnp.int32),
        pltpu.VMEM((32768,), jnp.float32),
        pltpu.VMEM((BS, 256), jnp.float32),
        pltpu.SemaphoreType.DMA,
        pltpu.SemaphoreType.DMA,
    ],
    compiler_params=pltpu.CompilerParams(needs_layout_passes=False),
)


def _maps_body(part_ref, maps_ref):
    p = part_ref[...]
    h = p[0] + p[1] + p[2] + p[3]
    all_sum = jnp.sum(h, axis=1, keepdims=True)
    thr = 10.0 * all_sum / 256.0
    total_extra = jnp.sum(jnp.maximum(h - thr, 0.0), axis=1, keepdims=True)
    mean_extra = total_extra / 256.0
    cliph = jnp.floor(jnp.minimum(h, thr) + mean_extra)
    ri = lax.broadcasted_iota(jnp.int32, (256, 256), 0)
    cj = lax.broadcasted_iota(jnp.int32, (256, 256), 1)
    tri = (ri <= cj).astype(jnp.float32)
    cdf = jnp.dot(cliph, tri, preferred_element_type=jnp.float32)
    maps = jnp.mod(jnp.floor(cdf * (255.0 / 65536.0)), 256.0)
    shifted = jnp.concatenate([maps[8:], maps[56:64]], axis=0)
    lo = lax.bitcast_convert_type(maps.astype(jnp.bfloat16), jnp.uint16)
    hi = lax.bitcast_convert_type(shifted.astype(jnp.bfloat16), jnp.uint16)
    maps_ref[...] = lo.astype(jnp.int32) | (hi.astype(jnp.int32) << 16)


_maps_call = pl.pallas_call(
    _maps_body,
    out_shape=jax.ShapeDtypeStruct((64, 256), jnp.int32),
)


def _interp_body(img_hbm, part_hbm, out_hbm, mapsv, imgbuf, outbuf,
                 pbuf, mbuf, hibuf, pkbuf, shf, shp,
                 semm, si0, si1, so0, so1):
    ci = lax.axis_index("c")
    si = lax.axis_index("s")
    w = si * 2 + ci
    row0 = w * RPW
    n_ch = RPW // CH
    sin = (si0, si1)
    sout = (so0, so1)
    lane = lax.iota(jnp.int32, LANES)
    lanef = lane.astype(jnp.float32) * (1.0 / 256.0)
    y1_seg = [lanef + (_SEG_STARTS[s] / 256.0 - (s + 0.5)) for s in range(7)]

    hin = [None, None]
    hout = [None, None]
    hin[0] = pltpu.async_copy(
        img_hbm.at[pl.ds(row0, CH)], imgbuf.at[0], sin[0])

    zeros = jnp.zeros((LANES,), jnp.float32)
    si4 = si * 4
    for k in range(4):
        pltpu.sync_copy(part_hbm.at[k, pl.ds(si4, 4)], pbuf.at[k])
    for bi in range(4):
        for c in range(16):
            ds16 = pl.ds(c * LANES, LANES)
            hc = (pbuf[0, bi, ds16] + pbuf[1, bi, ds16]
                  + pbuf[2, bi, ds16] + pbuf[3, bi, ds16])
            hibuf[pl.ds(bi * 256 + c * LANES, LANES)] = hc
        acc = zeros
        for c in range(16):
            acc = acc + hibuf[pl.ds(bi * 256 + c * LANES, LANES)]
        all_sum = jnp.sum(acc)
        thrv = lax.broadcast(all_sum, (LANES,)) * 10.0 / 256.0
        acce = zeros
        for c in range(16):
            acce = acce + jnp.maximum(
                hibuf[pl.ds(bi * 256 + c * LANES, LANES)] - thrv, 0.0)
        mev = lax.broadcast(jnp.sum(acce), (LANES,)) * (1.0 / 256.0)
        carry = zeros
        for c in range(16):
            cl = jnp.minimum(
                hibuf[pl.ds(bi * 256 + c * LANES, LANES)], thrv) + mev
            cli = cl.astype(jnp.int32).astype(jnp.float32)
            cs = plsc.cumsum(cli) + carry
            carry = lax.broadcast(jnp.max(cs), (LANES,))
            mp = (cs * (255.0 / 65536.0)).astype(jnp.int32)
            mbuf[pl.ds(bi * 256 + c * LANES, LANES)] = (
                (mp & 255).astype(jnp.float32))
    pltpu.sync_copy(mbuf, shf.at[pl.ds(si4 * 256, 1024)])
    plsc.subcore_barrier()
    histart = jnp.where(si < 14, si4 + 8, si4)
    pltpu.sync_copy(shf.at[pl.ds(histart * 256, 1024)], hibuf)
    for k in range(4):
        for c in range(16):
            ds16 = pl.ds(k * 256 + c * LANES, LANES)
            pk = plsc.bitcast(
                plsc.pack(mbuf[ds16], hibuf[ds16],
                          format=plsc.PackFormat.INTERLEAVED), jnp.int32)
            pkbuf[ds16] = pk
    pltpu.sync_copy(pkbuf, shp.at[pl.ds(si4 * 256, 1024)])
    plsc.subcore_barrier()
    pltpu.sync_copy(shp, mapsv)

    for ch in range(n_ch):
        par = ch & 1
        i0 = row0 + ch * CH
        if ch + 1 < n_ch:
            hin[1 - par] = pltpu.async_copy(
                img_hbm.at[pl.ds(row0 + (ch + 1) * CH, CH)],
                imgbuf.at[1 - par], sin[1 - par])
        hin[par].wait()
        if ch >= 2:
            hout[par].wait()

        def row_body(r8, _, par=par, i0=i0):
            i = i0 + r8
            r = jnp.maximum(i - 128, 0) >> 8
            rv = lax.broadcast(r * 2048, (LANES,))
            x1s = i - (r * 256 + 128)
            x1v = lax.broadcast(x1s, (LANES,)).astype(jnp.float32) * (1.0 / 256.0)
            redge = lax.broadcast(i >= 1920, (LANES,))
            x1v = jnp.where(redge, jnp.zeros((LANES,), jnp.float32), x1v)
            ex1 = 1.0 - x1v

            for seg in range(BS):
                start = _SEG_STARTS[seg]
                rvc0 = rv + seg * 256
                if seg < 7:
                    rvc1 = rv + (seg + 1) * 256

                    def run_body(t, y1v, start=start, rvc0=rvc0,
                                 rvc1=rvc1, par=par):
                        jb = start + t * LANES
                        v = imgbuf[par, r8, pl.ds(jb, LANES)]
                        g0 = plsc.load_gather(mapsv, [v + rvc0])
                        g1 = plsc.load_gather(mapsv, [v + rvc1])
                        lu, lb = plsc.unpack(
                            plsc.bitcast(g0, jnp.bfloat16),
                            format=plsc.PackFormat.INTERLEAVED)
                        ru, rb = plsc.unpack(
                            plsc.bitcast(g1, jnp.bfloat16),
                            format=plsc.PackFormat.INTERLEAVED)
                        t0 = ex1 * lu + x1v * lb
                        t1 = ex1 * ru + x1v * rb
                        res = (1.0 - y1v) * t0 + y1v * t1
                        q = res.astype(jnp.int32) & 255
                        outbuf[par, r8, pl.ds(jb, LANES)] = q.astype(jnp.float32)
                        return y1v + (LANES / 256.0)

                    plsc.parallel_loop(0, _SEG_RUNS[seg], unroll=4,
                                       carry=y1_seg[seg])(run_body)
                else:
                    def run_body7(t, start=start, rvc0=rvc0, par=par):
                        jb = start + t * LANES
                        v = imgbuf[par, r8, pl.ds(jb, LANES)]
                        g0 = plsc.load_gather(mapsv, [v + rvc0])
                        lu, lb = plsc.unpack(
                            plsc.bitcast(g0, jnp.bfloat16),
                            format=plsc.PackFormat.INTERLEAVED)
                        res = ex1 * lu + x1v * lb
                        q = res.astype(jnp.int32) & 255
                        outbuf[par, r8, pl.ds(jb, LANES)] = q.astype(jnp.float32)

                    plsc.parallel_loop(0, _SEG_RUNS[seg], unroll=4)(run_body7)
            return 0

        lax.fori_loop(0, CH, row_body, 0)
        hout[par] = pltpu.async_copy(
            outbuf.at[par], out_hbm.at[pl.ds(i0, CH)], sout[par])

    hout[(n_ch - 2) & 1].wait()
    hout[(n_ch - 1) & 1].wait()


_interp_kernel = pl.kernel(
    _interp_body,
    out_type=jax.ShapeDtypeStruct((M, M), jnp.float32),
    mesh=plsc.VectorSubcoreMesh(core_axis_name="c", subcore_axis_name="s"),
    scratch_types=[
        pltpu.VMEM((16384,), jnp.int32),
        pltpu.VMEM((2, CH, 2048), jnp.int32),
        pltpu.VMEM((2, CH, 2048), jnp.float32),
        pltpu.VMEM((4, 4, 256), jnp.float32),
        pltpu.VMEM((1024,), jnp.float32),
        pltpu.VMEM((1024,), jnp.float32),
        pltpu.VMEM((1024,), jnp.int32),
        pltpu.VMEM_SHARED((16384,), jnp.float32),
        pltpu.VMEM_SHARED((16384,), jnp.int32),
        pltpu.SemaphoreType.DMA,
        pltpu.SemaphoreType.DMA,
        pltpu.SemaphoreType.DMA,
        pltpu.SemaphoreType.DMA,
        pltpu.SemaphoreType.DMA,
    ],
    compiler_params=pltpu.CompilerParams(needs_layout_passes=False),
)


@jax.jit
def _clahe(img):
    partials = _hist_kernel(img)
    return _interp_kernel(img, partials)


def kernel(img_arr, level, blocks):
    return _clahe(img_arr.astype(jnp.int32))


# speedup vs baseline: 1.0113x; 1.0000x over previous
import functools
import jax
import jax.numpy as jnp
from jax import lax
from jax.experimental import pallas as pl
from jax.experimental.pallas import tpu as pltpu
from jax.experimental.pallas import tpu_sc as plsc

M = 2048
BS = 8
BM = M // BS
NW = 32
RPW = M // NW
CH = 8
LANES = 16

_SEG_STARTS = (0, 384, 640, 896, 1152, 1408, 1664, 1920)
_SEG_RUNS = (24, 16, 16, 16, 16, 16, 16, 8)


def _hist_body(img_hbm, part_hbm, imgbuf, hist, redbuf, sem0, sem1):
    ci = lax.axis_index("c")
    si = lax.axis_index("s")
    w = si * 2 + ci
    row0 = w * RPW
    lane = lax.iota(jnp.int32, LANES)
    laneoff = lane * 2048
    ones = jnp.ones((LANES,), jnp.float32)
    zeros = jnp.zeros((LANES,), jnp.float32)
    sems = (sem0, sem1)
    n_ch = RPW // CH

    handles = [None, None]
    handles[0] = pltpu.async_copy(
        img_hbm.at[pl.ds(row0, CH)], imgbuf.at[0], sems[0])

    def zero_body(t):
        hist[pl.ds(t * LANES, LANES)] = zeros

    plsc.parallel_loop(0, 32768 // LANES)(zero_body)

    for ch in range(n_ch):
        par = ch & 1
        if ch + 1 < n_ch:
            handles[1 - par] = pltpu.async_copy(
                img_hbm.at[pl.ds(row0 + (ch + 1) * CH, CH)],
                imgbuf.at[1 - par], sems[1 - par])
        handles[par].wait()

        def rs_body(t, par=par):
            row = t >> 3
            col0 = (t & 7) << 8
            svec = laneoff + col0
            for k in range(16):
                v = imgbuf[par, row, pl.ds(col0 + k * LANES, LANES)]
                plsc.addupdate_scatter(hist, [v + svec], ones)

        plsc.parallel_loop(0, CH * BS, unroll=1)(rs_body)

    for seg in range(BS):
        def red_body(c16, _):
            base = seg * 256 + c16 * LANES
            acc = hist[pl.ds(base, LANES)]
            for k in range(1, LANES):
                acc = acc + hist[pl.ds(k * 2048 + base, LANES)]
            redbuf[seg, pl.ds(c16 * LANES, LANES)] = acc
            return 0

        lax.fori_loop(0, 256 // LANES, red_body, 0)

    pltpu.sync_copy(redbuf, part_hbm.at[w % 4, pl.ds((w // 4) * BS, BS)])


_hist_kernel = pl.kernel(
    _hist_body,
    out_type=jax.ShapeDtypeStruct((4, 64, 256), jnp.float32),
    mesh=plsc.VectorSubcoreMesh(core_axis_name="c", subcore_axis_name="s"),
    scratch_types=[
        pltpu.VMEM((2, CH, 2048), jnp.int32),
        pltpu.VMEM((32768,), jnp.float32),
        pltpu.VMEM((BS, 256), jnp.float32),
        pltpu.SemaphoreType.DMA,
        pltpu.SemaphoreType.DMA,
    ],
    compiler_params=pltpu.CompilerParams(needs_layout_passes=False),
)


def _maps_body(part_ref, maps_ref):
    p = part_ref[...]
    h = p[0] + p[1] + p[2] + p[3]
    all_sum = jnp.sum(h, axis=1, keepdims=True)
    thr = 10.0 * all_sum / 256.0
    total_extra = jnp.sum(jnp.maximum(h - thr, 0.0), axis=1, keepdims=True)
    mean_extra = total_extra / 256.0
    cliph = jnp.floor(jnp.minimum(h, thr) + mean_extra)
    ri = lax.broadcasted_iota(jnp.int32, (256, 256), 0)
    cj = lax.broadcasted_iota(jnp.int32, (256, 256), 1)
    tri = (ri <= cj).astype(jnp.float32)
    cdf = jnp.dot(cliph, tri, preferred_element_type=jnp.float32)
    maps = jnp.mod(jnp.floor(cdf * (255.0 / 65536.0)), 256.0)
    shifted = jnp.concatenate([maps[8:], maps[56:64]], axis=0)
    lo = lax.bitcast_convert_type(maps.astype(jnp.bfloat16), jnp.uint16)
    hi = lax.bitcast_convert_type(shifted.astype(jnp.bfloat16), jnp.uint16)
    maps_ref[...] = lo.astype(jnp.int32) | (hi.astype(jnp.int32) << 16)


_maps_call = pl.pallas_call(
    _maps_body,
    out_shape=jax.ShapeDtypeStruct((64, 256), jnp.int32),
)


def _interp_body(img_hbm, part_hbm, out_hbm, mapsv, imgbuf, outbuf,
                 pbuf, mbuf, hibuf, pkbuf, shf, shp,
                 semm, si0, si1, so0, so1):
    ci = lax.axis_index("c")
    si = lax.axis_index("s")
    w = si * 2 + ci
    row0 = w * RPW
    n_ch = RPW // CH
    sin = (si0, si1)
    sout = (so0, so1)
    lane = lax.iota(jnp.int32, LANES)
    lanef = lane.astype(jnp.float32) * (1.0 / 256.0)
    y1_seg = [lanef + (_SEG_STARTS[s] / 256.0 - (s + 0.5)) for s in range(7)]

    hin = [None, None]
    hout = [None, None]
    hin[0] = pltpu.async_copy(
        img_hbm.at[pl.ds(row0, CH)], imgbuf.at[0], sin[0])

    zeros = jnp.zeros((LANES,), jnp.float32)
    si4 = si * 4
    for k in range(4):
        pltpu.sync_copy(part_hbm.at[k, pl.ds(si4, 4)], pbuf.at[k])
    for bi in range(4):
        for c in range(16):
            ds16 = pl.ds(c * LANES, LANES)
            hc = (pbuf[0, bi, ds16] + pbuf[1, bi, ds16]
                  + pbuf[2, bi, ds16] + pbuf[3, bi, ds16])
            hibuf[pl.ds(bi * 256 + c * LANES, LANES)] = hc
        acc = zeros
        for c in range(16):
            acc = acc + hibuf[pl.ds(bi * 256 + c * LANES, LANES)]
        all_sum = jnp.sum(acc)
        thrv = lax.broadcast(all_sum, (LANES,)) * 10.0 / 256.0
        acce = zeros
        for c in range(16):
            acce = acce + jnp.maximum(
                hibuf[pl.ds(bi * 256 + c * LANES, LANES)] - thrv, 0.0)
        mev = lax.broadcast(jnp.sum(acce), (LANES,)) * (1.0 / 256.0)
        carry = zeros
        for c in range(16):
            cl = jnp.minimum(
                hibuf[pl.ds(bi * 256 + c * LANES, LANES)], thrv) + mev
            cli = cl.astype(jnp.int32).astype(jnp.float32)
            cs = plsc.cumsum(cli) + carry
            carry = lax.broadcast(jnp.max(cs), (LANES,))
            mp = (cs * (255.0 / 65536.0)).astype(jnp.int32)
            mbuf[pl.ds(bi * 256 + c * LANES, LANES)] = (
                (mp & 255).astype(jnp.float32))
    pltpu.sync_copy(mbuf, shf.at[pl.ds(si4 * 256, 1024)])
    plsc.subcore_barrier()
    histart = jnp.where(si < 14, si4 + 8, si4)
    pltpu.sync_copy(shf.at[pl.ds(histart * 256, 1024)], hibuf)
    for k in range(4):
        for c in range(16):
            ds16 = pl.ds(k * 256 + c * LANES, LANES)
            pk = plsc.bitcast(
                plsc.pack(mbuf[ds16], hibuf[ds16],
                          format=plsc.PackFormat.INTERLEAVED), jnp.int32)
            pkbuf[ds16] = pk
    pltpu.sync_copy(pkbuf, shp.at[pl.ds(si4 * 256, 1024)])
    plsc.subcore_barrier()
    pltpu.sync_copy(shp, mapsv)

    for ch in range(n_ch):
        par = ch & 1
        i0 = row0 + ch * CH
        if ch + 1 < n_ch:
            hin[1 - par] = pltpu.async_copy(
                img_hbm.at[pl.ds(row0 + (ch + 1) * CH, CH)],
                imgbuf.at[1 - par], sin[1 - par])
        hin[par].wait()
        if ch >= 2:
            hout[par].wait()

        def row_body(r8, _, par=par, i0=i0):
            i = i0 + r8
            r = jnp.maximum(i - 128, 0) >> 8
            rv = lax.broadcast(r * 2048, (LANES,))
            x1s = i - (r * 256 + 128)
            x1v = lax.broadcast(x1s, (LANES,)).astype(jnp.float32) * (1.0 / 256.0)
            redge = lax.broadcast(i >= 1920, (LANES,))
            x1v = jnp.where(redge, jnp.zeros((LANES,), jnp.float32), x1v)
            ex1 = 1.0 - x1v

            for seg in range(BS):
                start = _SEG_STARTS[seg]
                rvc0 = rv + seg * 256
                if seg < 7:
                    rvc1 = rv + (seg + 1) * 256

                    def run_body(t, y1v, start=start, rvc0=rvc0,
                                 rvc1=rvc1, par=par):
                        jb = start + t * LANES
                        v = imgbuf[par, r8, pl.ds(jb, LANES)]
                        g0 = plsc.load_gather(mapsv, [v + rvc0])
                        g1 = plsc.load_gather(mapsv, [v + rvc1])
                        lu, lb = plsc.unpack(
                            plsc.bitcast(g0, jnp.bfloat16),
                            format=plsc.PackFormat.INTERLEAVED)
                        ru, rb = plsc.unpack(
                            plsc.bitcast(g1, jnp.bfloat16),
                            format=plsc.PackFormat.INTERLEAVED)
                        t0 = ex1 * lu + x1v * lb
                        t1 = ex1 * ru + x1v * rb
                        res = (1.0 - y1v) * t0 + y1v * t1
                        q = res.astype(jnp.int32) & 255
                        outbuf[par, r8, pl.ds(jb, LANES)] = q.astype(jnp.float32)
                        return y1v + (LANES / 256.0)

                    plsc.parallel_loop(0, _SEG_RUNS[seg], unroll=4,
                                       carry=y1_seg[seg])(run_body)
                else:
                    def run_body7(t, start=start, rvc0=rvc0, par=par):
                        jb = start + t * LANES
                        v = imgbuf[par, r8, pl.ds(jb, LANES)]
                        g0 = plsc.load_gather(mapsv, [v + rvc0])
                        lu, lb = plsc.unpack(
                            plsc.bitcast(g0, jnp.bfloat16),
                            format=plsc.PackFormat.INTERLEAVED)
                        res = ex1 * lu + x1v * lb
                        q = res.astype(jnp.int32) & 255
                        outbuf[par, r8, pl.ds(jb, LANES)] = q.astype(jnp.float32)

                    plsc.parallel_loop(0, _SEG_RUNS[seg], unroll=4)(run_body7)
            return 0

        lax.fori_loop(0, CH, row_body, 0)
        hout[par] = pltpu.async_copy(
            outbuf.at[par], out_hbm.at[pl.ds(i0, CH)], sout[par])

    hout[(n_ch - 2) & 1].wait()
    hout[(n_ch - 1) & 1].wait()


_interp_kernel = pl.kernel(
    _interp_body,
    out_type=jax.ShapeDtypeStruct((M, M), jnp.float32),
    mesh=plsc.VectorSubcoreMesh(core_axis_name="c", subcore_axis_name="s"),
    scratch_types=[
        pltpu.VMEM((16384,), jnp.int32),
        pltpu.VMEM((2, CH, 2048), jnp.int32),
        pltpu.VMEM((2, CH, 2048), jnp.float32),
        pltpu.VMEM((4, 4, 256), jnp.float32),
        pltpu.VMEM((1024,), jnp.float32),
        pltpu.VMEM((1024,), jnp.float32),
        pltpu.VMEM((1024,), jnp.int32),
        pltpu.VMEM_SHARED((16384,), jnp.float32),
        pltpu.VMEM_SHARED((16384,), jnp.int32),
        pltpu.SemaphoreType.DMA,
        pltpu.SemaphoreType.DMA,
        pltpu.SemaphoreType.DMA,
        pltpu.SemaphoreType.DMA,
        pltpu.SemaphoreType.DMA,
    ],
    compiler_params=pltpu.CompilerParams(needs_layout_passes=False),
)


@jax.jit
def _clahe(img):
    partials = _hist_kernel(img)
    return _interp_kernel(img, partials)


def kernel(img_arr, level, blocks):
    return _clahe(img_arr.astype(jnp.int32))


# speedup vs baseline: 1.0339x; 1.0224x over previous
import functools
import jax
import jax.numpy as jnp
from jax import lax
from jax.experimental import pallas as pl
from jax.experimental.pallas import tpu as pltpu
from jax.experimental.pallas import tpu_sc as plsc

M = 2048
BS = 8
BM = M // BS
NW = 32
RPW = M // NW
CH = 8
LANES = 16

_SEG_STARTS = (0, 384, 640, 896, 1152, 1408, 1664, 1920)
_SEG_RUNS = (24, 16, 16, 16, 16, 16, 16, 8)


CHH = 16


def _hist_body(img_hbm, part_hbm, imgbuf, hist, redbuf, sem0, sem1):
    ci = lax.axis_index("c")
    si = lax.axis_index("s")
    w = si * 2 + ci
    row0 = w * RPW
    lane = lax.iota(jnp.int32, LANES)
    laneoff = lane * 2048
    ones = jnp.ones((LANES,), jnp.float32)
    zeros = jnp.zeros((LANES,), jnp.float32)
    sems = (sem0, sem1)
    n_ch = RPW // CHH

    handles = [None, None]
    handles[0] = pltpu.async_copy(
        img_hbm.at[pl.ds(row0, CHH)], imgbuf.at[0], sems[0])

    def zero_body(t):
        hist[pl.ds(t * LANES, LANES)] = zeros

    plsc.parallel_loop(0, 32768 // LANES)(zero_body)

    for ch in range(n_ch):
        par = ch & 1
        if ch + 1 < n_ch:
            handles[1 - par] = pltpu.async_copy(
                img_hbm.at[pl.ds(row0 + (ch + 1) * CHH, CHH)],
                imgbuf.at[1 - par], sems[1 - par])
        handles[par].wait()

        def rs_body(t, par=par):
            row = t >> 3
            col0 = (t & 7) << 8
            svec = laneoff + col0
            for k in range(16):
                v = imgbuf[par, row, pl.ds(col0 + k * LANES, LANES)]
                plsc.addupdate_scatter(hist, [v + svec], ones)

        plsc.parallel_loop(0, CHH * BS, unroll=1)(rs_body)

    for seg in range(BS):
        def red_body(c16, _):
            base = seg * 256 + c16 * LANES
            acc = hist[pl.ds(base, LANES)]
            for k in range(1, LANES):
                acc = acc + hist[pl.ds(k * 2048 + base, LANES)]
            redbuf[seg, pl.ds(c16 * LANES, LANES)] = acc
            return 0

        lax.fori_loop(0, 256 // LANES, red_body, 0)

    pltpu.sync_copy(redbuf, part_hbm.at[w % 4, pl.ds((w // 4) * BS, BS)])


_hist_kernel = pl.kernel(
    _hist_body,
    out_type=jax.ShapeDtypeStruct((4, 64, 256), jnp.float32),
    mesh=plsc.VectorSubcoreMesh(core_axis_name="c", subcore_axis_name="s"),
    scratch_types=[
        pltpu.VMEM((2, CHH, 2048), jnp.int32),
        pltpu.VMEM((32768,), jnp.float32),
        pltpu.VMEM((BS, 256), jnp.float32),
        pltpu.SemaphoreType.DMA,
        pltpu.SemaphoreType.DMA,
    ],
    compiler_params=pltpu.CompilerParams(needs_layout_passes=False),
)


def _maps_body(part_ref, maps_ref):
    p = part_ref[...]
    h = p[0] + p[1] + p[2] + p[3]
    all_sum = jnp.sum(h, axis=1, keepdims=True)
    thr = 10.0 * all_sum / 256.0
    total_extra = jnp.sum(jnp.maximum(h - thr, 0.0), axis=1, keepdims=True)
    mean_extra = total_extra / 256.0
    cliph = jnp.floor(jnp.minimum(h, thr) + mean_extra)
    ri = lax.broadcasted_iota(jnp.int32, (256, 256), 0)
    cj = lax.broadcasted_iota(jnp.int32, (256, 256), 1)
    tri = (ri <= cj).astype(jnp.float32)
    cdf = jnp.dot(cliph, tri, preferred_element_type=jnp.float32)
    maps = jnp.mod(jnp.floor(cdf * (255.0 / 65536.0)), 256.0)
    shifted = jnp.concatenate([maps[8:], maps[56:64]], axis=0)
    lo = lax.bitcast_convert_type(maps.astype(jnp.bfloat16), jnp.uint16)
    hi = lax.bitcast_convert_type(shifted.astype(jnp.bfloat16), jnp.uint16)
    maps_ref[...] = lo.astype(jnp.int32) | (hi.astype(jnp.int32) << 16)


_maps_call = pl.pallas_call(
    _maps_body,
    out_shape=jax.ShapeDtypeStruct((64, 256), jnp.int32),
)


def _interp_body(img_hbm, part_hbm, out_hbm, mapsv, imgbuf, outbuf,
                 pbuf, mbuf, hibuf, pkbuf, shf, shp,
                 semm, si0, si1, so0, so1):
    ci = lax.axis_index("c")
    si = lax.axis_index("s")
    w = si * 2 + ci
    row0 = w * RPW
    n_ch = RPW // CH
    sin = (si0, si1)
    sout = (so0, so1)
    lane = lax.iota(jnp.int32, LANES)
    lanef = lane.astype(jnp.float32) * (1.0 / 256.0)
    y1_seg = [lanef + (_SEG_STARTS[s] / 256.0 - (s + 0.5)) for s in range(7)]

    hin = [None, None]
    hout = [None, None]
    hin[0] = pltpu.async_copy(
        img_hbm.at[pl.ds(row0, CH)], imgbuf.at[0], sin[0])

    zeros = jnp.zeros((LANES,), jnp.float32)
    si4 = si * 4
    for k in range(4):
        pltpu.sync_copy(part_hbm.at[k, pl.ds(si4, 4)], pbuf.at[k])
    for bi in range(4):
        for c in range(16):
            ds16 = pl.ds(c * LANES, LANES)
            hc = (pbuf[0, bi, ds16] + pbuf[1, bi, ds16]
                  + pbuf[2, bi, ds16] + pbuf[3, bi, ds16])
            hibuf[pl.ds(bi * 256 + c * LANES, LANES)] = hc
        acc = zeros
        for c in range(16):
            acc = acc + hibuf[pl.ds(bi * 256 + c * LANES, LANES)]
        all_sum = jnp.sum(acc)
        thrv = lax.broadcast(all_sum, (LANES,)) * 10.0 / 256.0
        acce = zeros
        for c in range(16):
            acce = acce + jnp.maximum(
                hibuf[pl.ds(bi * 256 + c * LANES, LANES)] - thrv, 0.0)
        mev = lax.broadcast(jnp.sum(acce), (LANES,)) * (1.0 / 256.0)
        carry = zeros
        for c in range(16):
            cl = jnp.minimum(
                hibuf[pl.ds(bi * 256 + c * LANES, LANES)], thrv) + mev
            cli = cl.astype(jnp.int32).astype(jnp.float32)
            cs = plsc.cumsum(cli) + carry
            carry = lax.broadcast(jnp.max(cs), (LANES,))
            mp = (cs * (255.0 / 65536.0)).astype(jnp.int32)
            mbuf[pl.ds(bi * 256 + c * LANES, LANES)] = (
                (mp & 255).astype(jnp.float32))
    pltpu.sync_copy(mbuf, shf.at[pl.ds(si4 * 256, 1024)])
    plsc.subcore_barrier()
    histart = jnp.where(si < 14, si4 + 8, si4)
    pltpu.sync_copy(shf.at[pl.ds(histart * 256, 1024)], hibuf)
    for k in range(4):
        for c in range(16):
            ds16 = pl.ds(k * 256 + c * LANES, LANES)
            pk = plsc.bitcast(
                plsc.pack(mbuf[ds16], hibuf[ds16],
                          format=plsc.PackFormat.INTERLEAVED), jnp.int32)
            pkbuf[ds16] = pk
    pltpu.sync_copy(pkbuf, shp.at[pl.ds(si4 * 256, 1024)])
    plsc.subcore_barrier()
    pltpu.sync_copy(shp, mapsv)

    for ch in range(n_ch):
        par = ch & 1
        i0 = row0 + ch * CH
        if ch + 1 < n_ch:
            hin[1 - par] = pltpu.async_copy(
                img_hbm.at[pl.ds(row0 + (ch + 1) * CH, CH)],
                imgbuf.at[1 - par], sin[1 - par])
        hin[par].wait()
        if ch >= 2:
            hout[par].wait()

        def row_body(r8, _, par=par, i0=i0):
            i = i0 + r8
            r = jnp.maximum(i - 128, 0) >> 8
            rv = lax.broadcast(r * 2048, (LANES,))
            x1s = i - (r * 256 + 128)
            x1v = lax.broadcast(x1s, (LANES,)).astype(jnp.float32) * (1.0 / 256.0)
            redge = lax.broadcast(i >= 1920, (LANES,))
            x1v = jnp.where(redge, jnp.zeros((LANES,), jnp.float32), x1v)
            ex1 = 1.0 - x1v

            for seg in range(BS):
                start = _SEG_STARTS[seg]
                rvc0 = rv + seg * 256
                if seg < 7:
                    rvc1 = rv + (seg + 1) * 256

                    def run_body(t, y1v, start=start, rvc0=rvc0,
                                 rvc1=rvc1, par=par):
                        jb = start + t * LANES
                        v = imgbuf[par, r8, pl.ds(jb, LANES)]
                        g0 = plsc.load_gather(mapsv, [v + rvc0])
                        g1 = plsc.load_gather(mapsv, [v + rvc1])
                        lu, lb = plsc.unpack(
                            plsc.bitcast(g0, jnp.bfloat16),
                            format=plsc.PackFormat.INTERLEAVED)
                        ru, rb = plsc.unpack(
                            plsc.bitcast(g1, jnp.bfloat16),
                            format=plsc.PackFormat.INTERLEAVED)
                        t0 = ex1 * lu + x1v * lb
                        t1 = ex1 * ru + x1v * rb
                        res = (1.0 - y1v) * t0 + y1v * t1
                        q = res.astype(jnp.int32) & 255
                        outbuf[par, r8, pl.ds(jb, LANES)] = q.astype(jnp.float32)
                        return y1v + (LANES / 256.0)

                    plsc.parallel_loop(0, _SEG_RUNS[seg], unroll=4,
                                       carry=y1_seg[seg])(run_body)
                else:
                    def run_body7(t, start=start, rvc0=rvc0, par=par):
                        jb = start + t * LANES
                        v = imgbuf[par, r8, pl.ds(jb, LANES)]
                        g0 = plsc.load_gather(mapsv, [v + rvc0])
                        lu, lb = plsc.unpack(
                            plsc.bitcast(g0, jnp.bfloat16),
                            format=plsc.PackFormat.INTERLEAVED)
                        res = ex1 * lu + x1v * lb
                        q = res.astype(jnp.int32) & 255
                        outbuf[par, r8, pl.ds(jb, LANES)] = q.astype(jnp.float32)

                    plsc.parallel_loop(0, _SEG_RUNS[seg], unroll=4)(run_body7)
            return 0

        lax.fori_loop(0, CH, row_body, 0)
        hout[par] = pltpu.async_copy(
            outbuf.at[par], out_hbm.at[pl.ds(i0, CH)], sout[par])

    hout[(n_ch - 2) & 1].wait()
    hout[(n_ch - 1) & 1].wait()


_interp_kernel = pl.kernel(
    _interp_body,
    out_type=jax.ShapeDtypeStruct((M, M), jnp.float32),
    mesh=plsc.VectorSubcoreMesh(core_axis_name="c", subcore_axis_name="s"),
    scratch_types=[
        pltpu.VMEM((16384,), jnp.int32),
        pltpu.VMEM((2, CH, 2048), jnp.int32),
        pltpu.VMEM((2, CH, 2048), jnp.float32),
        pltpu.VMEM((4, 4, 256), jnp.float32),
        pltpu.VMEM((1024,), jnp.float32),
        pltpu.VMEM((1024,), jnp.float32),
        pltpu.VMEM((1024,), jnp.int32),
        pltpu.VMEM_SHARED((16384,), jnp.float32),
        pltpu.VMEM_SHARED((16384,), jnp.int32),
        pltpu.SemaphoreType.DMA,
        pltpu.SemaphoreType.DMA,
        pltpu.SemaphoreType.DMA,
        pltpu.SemaphoreType.DMA,
        pltpu.SemaphoreType.DMA,
    ],
    compiler_params=pltpu.CompilerParams(needs_layout_passes=False),
)


@jax.jit
def _clahe(img):
    partials = _hist_kernel(img)
    return _interp_kernel(img, partials)


def kernel(img_arr, level, blocks):
    return _clahe(img_arr.astype(jnp.int32))
